# baseline (speedup 1.0000x reference)
import functools

import jax
import jax.numpy as jnp
from jax import lax
from jax.experimental import pallas as pl
from jax.experimental.pallas import tpu as pltpu
from jax.experimental.pallas import tpu_sc as plsc

N = 16384
C = 1000
QLO = (0, 256, 504, 760)
QHI = (256, 504, 760, 1000)
R0 = 256
R1 = 248
NC = 2
NS = 16
NW = NC * NS
CPW = N // NW
BLK = 256
NBLK = CPW // BLK
L = 16

_mesh = plsc.VectorSubcoreMesh(core_axis_name="c", subcore_axis_name="s")


@functools.partial(
    pl.kernel,
    out_type=jax.ShapeDtypeStruct((C, N), jnp.float32),
    mesh=_mesh,
    scratch_types=[
        pltpu.VMEM((CPW,), jnp.int32),
        pltpu.VMEM((R0, BLK), jnp.float32),
        pltpu.VMEM((R1, BLK), jnp.float32),
        pltpu.SemaphoreType.DMA,
        pltpu.SemaphoreType.DMA,
    ],
    compiler_params=pltpu.CompilerParams(needs_layout_passes=False),
)
def _onehot_sc(x_hbm, z0_hbm, z1_hbm, out_hbm, idx_v, buf_0, buf_1, sem_0, sem_1):
    wid = lax.axis_index("s") * NC + lax.axis_index("c")
    base = wid * CPW
    init_0 = pltpu.async_copy(z0_hbm, buf_0, sem_0)
    init_1 = pltpu.async_copy(z1_hbm, buf_1, sem_1)
    pltpu.sync_copy(x_hbm.at[pl.ds(base, CPW)], idx_v)

    ones = jnp.full((L,), 1.0, jnp.float32)
    zeros = jnp.zeros((L,), jnp.float32)
    cols = lax.iota(jnp.int32, L)

    def scatter(buf, q, b, val):
        lo, hi = QLO[q], QHI[q]
        for j in range(BLK // L):
            xv = idx_v[pl.ds(b * BLK + j * L, L)]
            rv = xv - lo
            mask = (xv >= lo) & (xv < hi)
            plsc.store_scatter(buf, [rv, cols + j * L], val, mask=mask)

    bufs = (buf_0, buf_1)
    sems = (sem_0, sem_1)
    pending = [init_0, init_1]
    prev_unit = [None, None]
    for b in range(NBLK):
        for q in range(4):
            p = q % 2
            buf, sem = bufs[p], sems[p]
            pending[p].wait()
            if prev_unit[p] is not None:
                pb, pq = prev_unit[p]
                scatter(buf, pq, pb, zeros)
            scatter(buf, q, b, ones)
            nrows = QHI[q] - QLO[q]
            src = buf if nrows == buf.shape[0] else buf.at[pl.ds(0, nrows), :]
            pending[p] = pltpu.async_copy(
                src,
                out_hbm.at[pl.ds(QLO[q], nrows), pl.ds(base + b * BLK, BLK)],
                sem,
            )
            prev_unit[p] = (b, q)
    pending[0].wait()
    pending[1].wait()


def kernel(x):
    z0 = jnp.zeros((R0, BLK), jnp.float32)
    z1 = jnp.zeros((R1, BLK), jnp.float32)
    return _onehot_sc(x.astype(jnp.int32), z0, z1).T


# speedup vs baseline: 1.0377x; 1.0377x over previous
import functools

import jax
import jax.numpy as jnp
from jax import lax
from jax.experimental import pallas as pl
from jax.experimental.pallas import tpu as pltpu
from jax.experimental.pallas import tpu_sc as plsc

N = 16384
C = 1000
CA = 504
CB = C - CA
NC = 2
NS = 16
NW = NC * NS
CPW = N // NW
BLK = 128
NBLK = CPW // BLK
L = 16

_mesh = plsc.VectorSubcoreMesh(core_axis_name="c", subcore_axis_name="s")


@functools.partial(
    pl.kernel,
    out_type=jax.ShapeDtypeStruct((C, N), jnp.float32),
    mesh=_mesh,
    scratch_types=[
        pltpu.VMEM((CPW,), jnp.int32),
        pltpu.VMEM((CA, BLK), jnp.float32),
        pltpu.VMEM((CB, BLK), jnp.float32),
        pltpu.SemaphoreType.DMA,
        pltpu.SemaphoreType.DMA,
    ],
    compiler_params=pltpu.CompilerParams(
        needs_layout_passes=False,
        skip_device_barrier=True,
        disable_bounds_checks=True,
        disable_semaphore_checks=True,
    ),
)
def _onehot_sc(x_hbm, za_hbm, zb_hbm, out_hbm, idx_v, buf_a, buf_b, sem_a, sem_b):
    wid = lax.axis_index("s") * NC + lax.axis_index("c")
    base = wid * CPW
    init_a = pltpu.async_copy(za_hbm, buf_a, sem_a)
    init_b = pltpu.async_copy(zb_hbm, buf_b, sem_b)
    pltpu.sync_copy(x_hbm.at[pl.ds(base, CPW)], idx_v)

    ones = jnp.full((L,), 1.0, jnp.float32)
    zeros = jnp.zeros((L,), jnp.float32)
    cols = lax.iota(jnp.int32, L)

    def scatter(buf, row0, nrows, b, val):
        for j in range(BLK // L):
            xv = idx_v[pl.ds(b * BLK + j * L, L)]
            rv = xv - row0
            mask = (xv >= row0) & (xv < row0 + nrows)
            plsc.store_scatter(buf, [rv, cols + j * L], val, mask=mask)

    prev_a = init_a
    prev_b = init_b
    for b in range(NBLK):
        prev_a.wait()
        if b > 0:
            scatter(buf_a, 0, CA, b - 1, zeros)
        scatter(buf_a, 0, CA, b, ones)
        prev_a = pltpu.async_copy(
            buf_a, out_hbm.at[pl.ds(0, CA), pl.ds(base + b * BLK, BLK)], sem_a
        )
        prev_b.wait()
        if b > 0:
            scatter(buf_b, CA, CB, b - 1, zeros)
        scatter(buf_b, CA, CB, b, ones)
        prev_b = pltpu.async_copy(
            buf_b, out_hbm.at[pl.ds(CA, CB), pl.ds(base + b * BLK, BLK)], sem_b
        )
    prev_a.wait()
    prev_b.wait()


def kernel(x):
    za = jnp.zeros((CA, BLK), jnp.float32)
    zb = jnp.zeros((CB, BLK), jnp.float32)
    return _onehot_sc(x.astype(jnp.int32), za, zb).T


# speedup vs baseline: 1.1006x; 1.0606x over previous
import functools

import jax
import jax.numpy as jnp
import numpy as np
from jax import lax
from jax.experimental import pallas as pl
from jax.experimental.pallas import tpu as pltpu
from jax.experimental.pallas import tpu_sc as plsc

N = 16384
C = 1000
QLO = (0, 256, 504, 760)
QHI = (256, 504, 760, 1000)
R0 = 256
R1 = 248
NC = 2
NS = 16
NW = NC * NS
CPW = N // NW
BLK = 128
NBLK = CPW // BLK
L = 16

_mesh = plsc.VectorSubcoreMesh(core_axis_name="c", subcore_axis_name="s")

_Z0 = np.zeros((R0, BLK), np.float32)
_Z1 = np.zeros((R1, BLK), np.float32)


@functools.partial(
    pl.kernel,
    out_type=jax.ShapeDtypeStruct((C, N), jnp.float32),
    mesh=_mesh,
    scratch_types=[
        pltpu.VMEM((CPW,), jnp.int32),
        pltpu.VMEM((R0, BLK), jnp.float32),
        pltpu.VMEM((R1, BLK), jnp.float32),
        pltpu.SemaphoreType.DMA,
        pltpu.SemaphoreType.DMA,
    ],
    compiler_params=pltpu.CompilerParams(
        needs_layout_passes=False,
        skip_device_barrier=True,
        disable_bounds_checks=True,
        disable_semaphore_checks=True,
    ),
)
def _onehot_sc(x_hbm, z0_hbm, z1_hbm, out_hbm, idx_v, buf_0, buf_1, sem_0, sem_1):
    wid = lax.axis_index("s") * NC + lax.axis_index("c")
    base = wid * CPW
    init_0 = pltpu.async_copy(z0_hbm, buf_0, sem_0)
    init_1 = pltpu.async_copy(z1_hbm, buf_1, sem_1)
    pltpu.sync_copy(x_hbm.at[pl.ds(base, CPW)], idx_v)

    ones = jnp.full((L,), 1.0, jnp.float32)
    zeros = jnp.zeros((L,), jnp.float32)
    cols = lax.iota(jnp.int32, L)

    def scatter(buf, q, b, val):
        lo, hi = QLO[q], QHI[q]
        for j in range(BLK // L):
            xv = idx_v[pl.ds(b * BLK + j * L, L)]
            rv = xv - lo
            mask = (xv >= lo) & (xv < hi)
            plsc.store_scatter(buf, [rv, cols + j * L], val, mask=mask)

    bufs = (buf_0, buf_1)
    sems = (sem_0, sem_1)
    pending = [init_0, init_1]
    prev_unit = [None, None]
    for b in range(NBLK):
        for q in range(4):
            p = q % 2
            buf, sem = bufs[p], sems[p]
            pending[p].wait()
            if prev_unit[p] is not None:
                pb, pq = prev_unit[p]
                scatter(buf, pq, pb, zeros)
            scatter(buf, q, b, ones)
            nrows = QHI[q] - QLO[q]
            src = buf if nrows == buf.shape[0] else buf.at[pl.ds(0, nrows), :]
            pending[p] = pltpu.async_copy(
                src,
                out_hbm.at[pl.ds(QLO[q], nrows), pl.ds(base + b * BLK, BLK)],
                sem,
            )
            prev_unit[p] = (b, q)
    pending[0].wait()
    pending[1].wait()


def kernel(x):
    return _onehot_sc(x.astype(jnp.int32), _Z0, _Z1).T
